# baseline (speedup 1.0000x reference)
import functools

import jax
import jax.numpy as jnp
from jax import lax
from jax.experimental import pallas as pl
from jax.experimental.pallas import tpu as pltpu
from jax.experimental.pallas import tpu_sc as plsc

NN = 10000
NP = 10240
NE = 320000
EPAD = 327680
DM = 128
NTIL = 16
RPT = NP // NTIL
CHUNK = 128
INNER = 8
IROWS = EPAD // CHUNK
IPT = IROWS // NTIL
OUTER = IPT // INNER
NBUF = 2
LOOK = 1
QROWS = 40
QN = IPT // QROWS
DEGW = 16
NSL = 16
BLK = 1024
NB1 = NP // BLK
NBLK = 2 * NP // BLK

_mesh = functools.partial(
    plsc.VectorSubcoreMesh, core_axis_name="c", subcore_axis_name="s")


def _sc_deg_body(dst2d, ones_hbm, deg_out, idx_v, ones_v, deg_sh):
    c = lax.axis_index("c")
    s = lax.axis_index("s")
    r0 = s * RPT
    pltpu.sync_copy(ones_hbm.at[pl.ds(0, CHUNK)], ones_v)
    pltpu.sync_copy(ones_hbm.at[pl.ds(r0, RPT)], deg_sh.at[pl.ds(r0, RPT)])
    plsc.subcore_barrier()
    row0 = c * IROWS + s * IPT

    def outer(g, carry):
        pltpu.sync_copy(dst2d.at[pl.ds(row0 + g * INNER, INNER)], idx_v)
        for j in range(INNER):
            pltpu.sync_copy(ones_v, deg_sh.at[idx_v.at[j]], add=True)
        return carry

    lax.fori_loop(0, OUTER, outer, 0)
    plsc.subcore_barrier()
    pltpu.sync_copy(deg_sh.at[pl.ds(r0, RPT)],
                    deg_out.at[pl.ds(c * NP + r0, RPT)])


def _sc_deg(dst2d, ones_hbm):
    return pl.kernel(
        _sc_deg_body,
        out_type=jax.ShapeDtypeStruct((2 * NP, DEGW), jnp.float32),
        mesh=_mesh(),
        scratch_types=[
            pltpu.VMEM((INNER, CHUNK), jnp.int32),
            pltpu.VMEM((CHUNK, DEGW), jnp.float32),
            pltpu.VMEM_SHARED((NP, DEGW), jnp.float32),
        ],
    )(dst2d, ones_hbm)


def _sc_scat_body(y_hbm, src2d, dst2d, z_out, idxs, idxd, rows, z_sh,
                  gsem, ssem):
    c = lax.axis_index("c")
    s = lax.axis_index("s")
    r0 = s * RPT
    base = c * NP
    pltpu.sync_copy(y_hbm.at[pl.ds(base + r0, RPT)], z_sh.at[pl.ds(r0, RPT)])
    row0 = c * IROWS + s * IPT
    plsc.subcore_barrier()

    def _gather(chunk, b):
        return pltpu.async_copy(y_hbm.at[idxs.at[chunk]], rows.at[b],
                                gsem.at[b])

    def _scat(chunk, b):
        return pltpu.async_copy(rows.at[b], z_sh.at[idxd.at[chunk]],
                                ssem.at[b], add=True)

    def window(q, carry):
        pltpu.sync_copy(src2d.at[pl.ds(row0 + q * QROWS, QROWS)], idxs)
        pltpu.sync_copy(dst2d.at[pl.ds(row0 + q * QROWS, QROWS)], idxd)
        for b in range(LOOK):
            _gather(b, b)

        def outer(w, carry2):
            for b in range(NBUF):
                ch = w * NBUF + b
                pltpu.make_async_copy(y_hbm.at[idxs.at[ch]], rows.at[b],
                                      gsem.at[b]).wait()
                _scat(ch, b)
                nb = (b + LOOK) % NBUF
                nc = ch + LOOK

                @pl.when(jnp.logical_and(nc >= NBUF, nc < QROWS))
                def _():
                    pltpu.make_async_copy(rows.at[nb],
                                          z_sh.at[idxd.at[ch]],
                                          ssem.at[nb]).wait()
                    _gather(nc, nb)

                @pl.when(jnp.logical_and(nc >= LOOK, nc < NBUF))
                def _():
                    _gather(nc, nb)

            return carry2

        lax.fori_loop(0, QROWS // NBUF, outer, 0)
        for b in range(NBUF):
            pltpu.make_async_copy(rows.at[b], z_sh.at[idxd.at[0]],
                                  ssem.at[b]).wait()
        return carry

    lax.fori_loop(0, QN, window, 0)
    plsc.subcore_barrier()
    pltpu.sync_copy(z_sh.at[pl.ds(r0, RPT)], z_out.at[pl.ds(base + r0, RPT)])


def _sc_scatter(y, src2d, dst2d):
    return pl.kernel(
        _sc_scat_body,
        out_type=jax.ShapeDtypeStruct((2 * NP, DM), jnp.float32),
        mesh=_mesh(),
        scratch_types=[
            pltpu.VMEM((QROWS, CHUNK), jnp.int32),
            pltpu.VMEM((QROWS, CHUNK), jnp.int32),
            pltpu.VMEM((NBUF, CHUNK, DM), jnp.float32),
            pltpu.VMEM_SHARED((NP, DM), jnp.float32),
            pltpu.SemaphoreType.DMA((NBUF,)),
            pltpu.SemaphoreType.DMA((NBUF,)),
        ],
    )(y, src2d, dst2d)


def _mm_body(x_ref, w_ref, y_ref):
    y_ref[...] = jnp.dot(x_ref[...], w_ref[...],
                         preferred_element_type=jnp.float32)


def _mm(x2, w):
    return pl.pallas_call(
        _mm_body,
        grid=(NBLK,),
        in_specs=[
            pl.BlockSpec((BLK, DM), lambda i: (i, 0)),
            pl.BlockSpec((DM, DM), lambda i: (0, 0)),
        ],
        out_specs=pl.BlockSpec((BLK, DM), lambda i: (i, 0)),
        out_shape=jax.ShapeDtypeStruct((2 * NP, DM), jnp.float32),
    )(x2, w)


def _scale_body(xw_ref, deg_ref, y_ref):
    dis = lax.rsqrt(deg_ref[:, 0:1])
    y_ref[...] = dis * xw_ref[...]


def _scale(xw, deg16):
    return pl.pallas_call(
        _scale_body,
        grid=(NBLK,),
        in_specs=[
            pl.BlockSpec((BLK, DM), lambda i: (i, 0)),
            pl.BlockSpec((BLK, DEGW), lambda i: (i, 0)),
        ],
        out_specs=pl.BlockSpec((BLK, DM), lambda i: (i, 0)),
        out_shape=jax.ShapeDtypeStruct((2 * NP, DM), jnp.float32),
    )(xw, deg16)


def _mid_body(z_ref, deg_ref, b_ref, w_ref, y_ref):
    dis = lax.rsqrt(deg_ref[:, 0:1])
    h = jax.nn.relu(dis * z_ref[...] + b_ref[...])
    y_ref[...] = dis * jnp.dot(
        h, w_ref[...], preferred_element_type=jnp.float32)


def _mid(z, deg16, b_row, w):
    return pl.pallas_call(
        _mid_body,
        grid=(NBLK,),
        in_specs=[
            pl.BlockSpec((BLK, DM), lambda i: (i, 0)),
            pl.BlockSpec((BLK, DEGW), lambda i: (i, 0)),
            pl.BlockSpec((1, DM), lambda i: (0, 0)),
            pl.BlockSpec((DM, DM), lambda i: (0, 0)),
        ],
        out_specs=pl.BlockSpec((BLK, DM), lambda i: (i, 0)),
        out_shape=jax.ShapeDtypeStruct((2 * NP, DM), jnp.float32),
    )(z, deg16, b_row, w)


def _valid_mask():
    row = pl.program_id(1) * BLK + lax.broadcasted_iota(
        jnp.int32, (BLK, 1), 0)
    return row < NN


def _fin_body(z_ref, deg_ref, b_ref, h_ref, cs_ref):
    h = lax.rsqrt(deg_ref[:, 0:1]) * z_ref[...] + b_ref[...]
    h_ref[...] = h

    @pl.when(pl.program_id(1) == 0)
    def _():
        cs_ref[...] = jnp.zeros_like(cs_ref)

    hm = jnp.where(_valid_mask(), h, 0.0)
    cs_ref[...] += jnp.broadcast_to(jnp.sum(hm, axis=0, keepdims=True),
                                    (8, DM))


def _fin(z, deg16, b_row):
    return pl.pallas_call(
        _fin_body,
        grid=(2, NB1),
        in_specs=[
            pl.BlockSpec((BLK, DM), lambda g, i: (g * NB1 + i, 0)),
            pl.BlockSpec((BLK, DEGW), lambda g, i: (g * NB1 + i, 0)),
            pl.BlockSpec((1, DM), lambda g, i: (0, 0)),
        ],
        out_specs=[
            pl.BlockSpec((BLK, DM), lambda g, i: (g * NB1 + i, 0)),
            pl.BlockSpec((8, DM), lambda g, i: (g, 0)),
        ],
        out_shape=[
            jax.ShapeDtypeStruct((2 * NP, DM), jnp.float32),
            jax.ShapeDtypeStruct((16, DM), jnp.float32),
        ],
    )(z, deg16, b_row)


def _att_body(h_ref, cs_ref, wa_ref, g_ref):
    gc = jnp.tanh(jnp.dot(cs_ref[0:1, :] * (1.0 / NN), wa_ref[...],
                          preferred_element_type=jnp.float32))
    logits = jnp.sum(h_ref[...] * gc, axis=1, keepdims=True)
    aw = jnp.where(_valid_mask(), jax.nn.sigmoid(logits), 0.0)

    @pl.when(pl.program_id(1) == 0)
    def _():
        g_ref[...] = jnp.zeros_like(g_ref)

    g_ref[...] += jnp.broadcast_to(
        jnp.sum(h_ref[...] * aw, axis=0, keepdims=True), (8, DM))


def _att(h3, cs, att_w):
    return pl.pallas_call(
        _att_body,
        grid=(2, NB1),
        in_specs=[
            pl.BlockSpec((BLK, DM), lambda g, i: (g * NB1 + i, 0)),
            pl.BlockSpec((8, DM), lambda g, i: (g, 0)),
            pl.BlockSpec((DM, DM), lambda g, i: (0, 0)),
        ],
        out_specs=pl.BlockSpec((8, DM), lambda g, i: (g, 0)),
        out_shape=jax.ShapeDtypeStruct((16, DM), jnp.float32),
    )(h3, cs, att_w)


def _head_body(g_ref, t2_ref, mt_ref, nb_ref, w1_ref, b1_ref, w2_ref, b2_ref,
               w3_ref, b3_ref, w4_ref, b4_ref, sw_ref, sb_ref, out_ref):
    gi = g_ref[0:1, :]
    gj = g_ref[1:2, :]
    u = jnp.dot(gi, t2_ref[...], preferred_element_type=jnp.float32)
    lane = lax.broadcasted_iota(jnp.int32, (1, NSL), 1)
    s1 = jnp.zeros((1, NSL), jnp.float32)
    for k in range(NSL):
        sk = jnp.sum(u[:, k * DM:(k + 1) * DM] * gj, axis=1, keepdims=True)
        s1 = s1 + jnp.where(lane == k, sk, 0.0)
    s2 = (jnp.dot(gi, mt_ref[0:DM, :], preferred_element_type=jnp.float32)
          + jnp.dot(gj, mt_ref[DM:2 * DM, :],
                    preferred_element_type=jnp.float32))
    sc = jnp.tanh(s1 + s2 + nb_ref[...])
    h = jax.nn.relu(jnp.dot(sc, w1_ref[...],
                            preferred_element_type=jnp.float32) + b1_ref[...])
    h = jax.nn.relu(jnp.dot(h, w2_ref[...],
                            preferred_element_type=jnp.float32) + b2_ref[...])
    h = jax.nn.relu(jnp.dot(h, w3_ref[...],
                            preferred_element_type=jnp.float32) + b3_ref[...])
    h = jax.nn.relu(jnp.dot(h, w4_ref[...],
                            preferred_element_type=jnp.float32) + b4_ref[...])
    out_ref[...] = jax.nn.sigmoid(
        jnp.dot(h, sw_ref[...], preferred_element_type=jnp.float32)
        + sb_ref[...])


def _head(g2, t2, mt, nb_row, mws):
    return pl.pallas_call(
        _head_body,
        out_shape=jax.ShapeDtypeStruct((1, 1), jnp.float32),
    )(g2, t2, mt, nb_row, *mws)


def kernel(x_i, edge_index_i, x_j, edge_index_j, W1, b1, W2, b2, W3, b3,
           att_W, ntn_T, ntn_M, ntn_b, mW1, mb1, mW2, mb2, mW3, mb3,
           mW4, mb4, sW, sb):
    ei = edge_index_i.astype(jnp.int32)
    ej = edge_index_j.astype(jnp.int32)
    pad = jnp.full((EPAD - NE,), NN, jnp.int32)
    src2d = jnp.concatenate(
        [ei[0], pad, ej[0] + NP, pad + NP]).reshape(2 * IROWS, CHUNK)
    dst2d = jnp.concatenate(
        [ei[1], pad, ej[1], pad]).reshape(2 * IROWS, CHUNK)
    ones16 = jnp.ones((NP, DEGW), jnp.float32)
    xpad = jnp.zeros((NP - NN, DM), jnp.float32)
    x2 = jnp.concatenate([x_i, xpad, x_j, xpad], axis=0)

    deg16 = _sc_deg(dst2d, ones16)
    xw1 = _mm(x2, W1)

    y = _scale(xw1, deg16)
    z = _sc_scatter(y, src2d, dst2d)
    y = _mid(z, deg16, b1.reshape(1, DM), W2)
    z = _sc_scatter(y, src2d, dst2d)
    y = _mid(z, deg16, b2.reshape(1, DM), W3)
    z = _sc_scatter(y, src2d, dst2d)
    h3, cs = _fin(z, deg16, b3.reshape(1, DM))
    g2 = _att(h3, cs, att_W)[::8]

    t2 = jnp.transpose(ntn_T, (1, 0, 2)).reshape(DM, NSL * DM)
    mt = ntn_M.T
    nb_row = ntn_b.reshape(1, NSL)
    mws = (mW1, mb1.reshape(1, -1), mW2, mb2.reshape(1, -1),
           mW3, mb3.reshape(1, -1), mW4, mb4.reshape(1, -1),
           sW, sb.reshape(1, 1))
    out = _head(g2, t2, mt, nb_row, mws)
    return out[0]


# speedup vs baseline: 1.0360x; 1.0360x over previous
import functools

import jax
import jax.numpy as jnp
from jax import lax
from jax.experimental import pallas as pl
from jax.experimental.pallas import tpu as pltpu
from jax.experimental.pallas import tpu_sc as plsc

NN = 10000
NP = 10240
NE = 320000
EPAD = 327680
DM = 128
NTIL = 16
RPT = NP // NTIL
CHUNK = 128
INNER = 8
IROWS = EPAD // CHUNK
IPT = IROWS // NTIL
OUTER = IPT // INNER
NBUF = 2
LOOK = 1
QROWS = 40
QN = IPT // QROWS
DEGW = 16
NSL = 16
BLK = 1024
NB1 = NP // BLK
NBLK = 2 * NP // BLK

_mesh = functools.partial(
    plsc.VectorSubcoreMesh, core_axis_name="c", subcore_axis_name="s")


def _sc_deg_body(dst2d, ones_hbm, deg_out, idx_v, ones_v, deg_sh):
    c = lax.axis_index("c")
    s = lax.axis_index("s")
    r0 = s * RPT
    pltpu.sync_copy(ones_hbm.at[pl.ds(0, CHUNK)], ones_v)
    pltpu.sync_copy(ones_hbm.at[pl.ds(r0, RPT)], deg_sh.at[pl.ds(r0, RPT)])
    plsc.subcore_barrier()
    row0 = c * IROWS + s * IPT

    def outer(g, carry):
        pltpu.sync_copy(dst2d.at[pl.ds(row0 + g * INNER, INNER)], idx_v)
        for j in range(INNER):
            pltpu.sync_copy(ones_v, deg_sh.at[idx_v.at[j]], add=True)
        return carry

    lax.fori_loop(0, OUTER, outer, 0)
    plsc.subcore_barrier()
    pltpu.sync_copy(deg_sh.at[pl.ds(r0, RPT)],
                    deg_out.at[pl.ds(c * NP + r0, RPT)])


def _sc_deg(dst2d, ones_hbm):
    return pl.kernel(
        _sc_deg_body,
        out_type=jax.ShapeDtypeStruct((2 * NP, DEGW), jnp.float32),
        mesh=_mesh(),
        scratch_types=[
            pltpu.VMEM((INNER, CHUNK), jnp.int32),
            pltpu.VMEM((CHUNK, DEGW), jnp.float32),
            pltpu.VMEM_SHARED((NP, DEGW), jnp.float32),
        ],
    )(dst2d, ones_hbm)


def _sc_scat_body(y_hbm, src2d, dst2d, z_out, idxs, idxd, rows, z_sh,
                  gsem, ssem):
    c = lax.axis_index("c")
    s = lax.axis_index("s")
    r0 = s * RPT
    base = c * NP
    pltpu.sync_copy(y_hbm.at[pl.ds(base + r0, RPT)], z_sh.at[pl.ds(r0, RPT)])
    row0 = c * IROWS + s * IPT
    plsc.subcore_barrier()

    def _gather(chunk, b):
        return pltpu.async_copy(y_hbm.at[idxs.at[chunk]], rows.at[b],
                                gsem.at[b])

    def _scat(chunk, b):
        return pltpu.async_copy(rows.at[b], z_sh.at[idxd.at[chunk]],
                                ssem.at[b], add=True)

    def window(q, carry):
        pltpu.sync_copy(src2d.at[pl.ds(row0 + q * QROWS, QROWS)], idxs)
        pltpu.sync_copy(dst2d.at[pl.ds(row0 + q * QROWS, QROWS)], idxd)
        for b in range(LOOK):
            _gather(b, b)

        def outer(w, carry2):
            for b in range(NBUF):
                ch = w * NBUF + b
                pltpu.make_async_copy(y_hbm.at[idxs.at[ch]], rows.at[b],
                                      gsem.at[b]).wait()
                _scat(ch, b)
                nb = (b + LOOK) % NBUF
                nc = ch + LOOK

                @pl.when(jnp.logical_and(nc >= NBUF, nc < QROWS))
                def _():
                    pltpu.make_async_copy(rows.at[nb],
                                          z_sh.at[idxd.at[ch]],
                                          ssem.at[nb]).wait()
                    _gather(nc, nb)

                @pl.when(jnp.logical_and(nc >= LOOK, nc < NBUF))
                def _():
                    _gather(nc, nb)

            return carry2

        lax.fori_loop(0, QROWS // NBUF, outer, 0)
        for b in range(NBUF):
            pltpu.make_async_copy(rows.at[b], z_sh.at[idxd.at[0]],
                                  ssem.at[b]).wait()
        return carry

    lax.fori_loop(0, QN, window, 0)
    plsc.subcore_barrier()
    pltpu.sync_copy(z_sh.at[pl.ds(r0, RPT)], z_out.at[pl.ds(base + r0, RPT)])


def _sc_scatter(y, src2d, dst2d):
    return pl.kernel(
        _sc_scat_body,
        out_type=jax.ShapeDtypeStruct((2 * NP, DM), jnp.float32),
        mesh=_mesh(),
        scratch_types=[
            pltpu.VMEM((QROWS, CHUNK), jnp.int32),
            pltpu.VMEM((QROWS, CHUNK), jnp.int32),
            pltpu.VMEM((NBUF, CHUNK, DM), jnp.float32),
            pltpu.VMEM_SHARED((NP, DM), jnp.float32),
            pltpu.SemaphoreType.DMA((NBUF,)),
            pltpu.SemaphoreType.DMA((NBUF,)),
        ],
    )(y, src2d, dst2d)


def _dis_body(deg_ref, out_ref):
    d = deg_ref[:, 0:1]
    out_ref[...] = jnp.broadcast_to(lax.rsqrt(d), (BLK, DM))


def _dis(deg16):
    return pl.pallas_call(
        _dis_body,
        grid=(NBLK,),
        in_specs=[pl.BlockSpec((BLK, DEGW), lambda i: (i, 0))],
        out_specs=pl.BlockSpec((BLK, DM), lambda i: (i, 0)),
        out_shape=jax.ShapeDtypeStruct((2 * NP, DM), jnp.float32),
    )(deg16)


def _first_body(x_ref, dis_ref, w_ref, y_ref):
    y_ref[...] = dis_ref[...] * jnp.dot(
        x_ref[...], w_ref[...], preferred_element_type=jnp.float32)


def _first(x2, disb, w):
    return pl.pallas_call(
        _first_body,
        grid=(NBLK,),
        in_specs=[
            pl.BlockSpec((BLK, DM), lambda i: (i, 0)),
            pl.BlockSpec((BLK, DM), lambda i: (i, 0)),
            pl.BlockSpec((DM, DM), lambda i: (0, 0)),
        ],
        out_specs=pl.BlockSpec((BLK, DM), lambda i: (i, 0)),
        out_shape=jax.ShapeDtypeStruct((2 * NP, DM), jnp.float32),
    )(x2, disb, w)


def _mid_body(z_ref, dis_ref, b_ref, w_ref, y_ref):
    h = jax.nn.relu(dis_ref[...] * z_ref[...] + b_ref[...])
    y_ref[...] = dis_ref[...] * jnp.dot(
        h, w_ref[...], preferred_element_type=jnp.float32)


def _mid(z, disb, b_row, w):
    return pl.pallas_call(
        _mid_body,
        grid=(NBLK,),
        in_specs=[
            pl.BlockSpec((BLK, DM), lambda i: (i, 0)),
            pl.BlockSpec((BLK, DM), lambda i: (i, 0)),
            pl.BlockSpec((1, DM), lambda i: (0, 0)),
            pl.BlockSpec((DM, DM), lambda i: (0, 0)),
        ],
        out_specs=pl.BlockSpec((BLK, DM), lambda i: (i, 0)),
        out_shape=jax.ShapeDtypeStruct((2 * NP, DM), jnp.float32),
    )(z, disb, b_row, w)


def _valid_mask():
    row = pl.program_id(1) * BLK + lax.broadcasted_iota(
        jnp.int32, (BLK, 1), 0)
    return row < NN


def _fin_body(z_ref, dis_ref, b_ref, h_ref, cs_ref):
    h = dis_ref[...] * z_ref[...] + b_ref[...]
    h_ref[...] = h

    @pl.when(pl.program_id(1) == 0)
    def _():
        cs_ref[...] = jnp.zeros_like(cs_ref)

    hm = jnp.where(_valid_mask(), h, 0.0)
    cs_ref[...] += jnp.broadcast_to(jnp.sum(hm, axis=0, keepdims=True),
                                    (8, DM))


def _fin(z, disb, b_row):
    return pl.pallas_call(
        _fin_body,
        grid=(2, NB1),
        in_specs=[
            pl.BlockSpec((BLK, DM), lambda g, i: (g * NB1 + i, 0)),
            pl.BlockSpec((BLK, DM), lambda g, i: (g * NB1 + i, 0)),
            pl.BlockSpec((1, DM), lambda g, i: (0, 0)),
        ],
        out_specs=[
            pl.BlockSpec((BLK, DM), lambda g, i: (g * NB1 + i, 0)),
            pl.BlockSpec((8, DM), lambda g, i: (g, 0)),
        ],
        out_shape=[
            jax.ShapeDtypeStruct((2 * NP, DM), jnp.float32),
            jax.ShapeDtypeStruct((16, DM), jnp.float32),
        ],
    )(z, disb, b_row)


def _att_body(h_ref, cs_ref, wa_ref, g_ref):
    gc = jnp.tanh(jnp.dot(cs_ref[0:1, :] * (1.0 / NN), wa_ref[...],
                          preferred_element_type=jnp.float32))
    logits = jnp.sum(h_ref[...] * gc, axis=1, keepdims=True)
    aw = jnp.where(_valid_mask(), jax.nn.sigmoid(logits), 0.0)

    @pl.when(pl.program_id(1) == 0)
    def _():
        g_ref[...] = jnp.zeros_like(g_ref)

    g_ref[...] += jnp.broadcast_to(
        jnp.sum(h_ref[...] * aw, axis=0, keepdims=True), (8, DM))


def _att(h3, cs, att_w):
    return pl.pallas_call(
        _att_body,
        grid=(2, NB1),
        in_specs=[
            pl.BlockSpec((BLK, DM), lambda g, i: (g * NB1 + i, 0)),
            pl.BlockSpec((8, DM), lambda g, i: (g, 0)),
            pl.BlockSpec((DM, DM), lambda g, i: (0, 0)),
        ],
        out_specs=pl.BlockSpec((8, DM), lambda g, i: (g, 0)),
        out_shape=jax.ShapeDtypeStruct((16, DM), jnp.float32),
    )(h3, cs, att_w)


def _head_body(g_ref, t2_ref, mt_ref, nb_ref, w1_ref, b1_ref, w2_ref, b2_ref,
               w3_ref, b3_ref, w4_ref, b4_ref, sw_ref, sb_ref, out_ref):
    gi = g_ref[0:1, :]
    gj = g_ref[1:2, :]
    u = jnp.dot(gi, t2_ref[...], preferred_element_type=jnp.float32)
    lane = lax.broadcasted_iota(jnp.int32, (1, NSL), 1)
    s1 = jnp.zeros((1, NSL), jnp.float32)
    for k in range(NSL):
        sk = jnp.sum(u[:, k * DM:(k + 1) * DM] * gj, axis=1, keepdims=True)
        s1 = s1 + jnp.where(lane == k, sk, 0.0)
    s2 = (jnp.dot(gi, mt_ref[0:DM, :], preferred_element_type=jnp.float32)
          + jnp.dot(gj, mt_ref[DM:2 * DM, :],
                    preferred_element_type=jnp.float32))
    sc = jnp.tanh(s1 + s2 + nb_ref[...])
    h = jax.nn.relu(jnp.dot(sc, w1_ref[...],
                            preferred_element_type=jnp.float32) + b1_ref[...])
    h = jax.nn.relu(jnp.dot(h, w2_ref[...],
                            preferred_element_type=jnp.float32) + b2_ref[...])
    h = jax.nn.relu(jnp.dot(h, w3_ref[...],
                            preferred_element_type=jnp.float32) + b3_ref[...])
    h = jax.nn.relu(jnp.dot(h, w4_ref[...],
                            preferred_element_type=jnp.float32) + b4_ref[...])
    out_ref[...] = jax.nn.sigmoid(
        jnp.dot(h, sw_ref[...], preferred_element_type=jnp.float32)
        + sb_ref[...])


def _head(g2, t2, mt, nb_row, mws):
    return pl.pallas_call(
        _head_body,
        out_shape=jax.ShapeDtypeStruct((1, 1), jnp.float32),
    )(g2, t2, mt, nb_row, *mws)


def kernel(x_i, edge_index_i, x_j, edge_index_j, W1, b1, W2, b2, W3, b3,
           att_W, ntn_T, ntn_M, ntn_b, mW1, mb1, mW2, mb2, mW3, mb3,
           mW4, mb4, sW, sb):
    ei = edge_index_i.astype(jnp.int32)
    ej = edge_index_j.astype(jnp.int32)
    pad = jnp.full((EPAD - NE,), NN, jnp.int32)
    src2d = jnp.concatenate(
        [ei[0], pad, ej[0] + NP, pad + NP]).reshape(2 * IROWS, CHUNK)
    dst2d = jnp.concatenate(
        [ei[1], pad, ej[1], pad]).reshape(2 * IROWS, CHUNK)
    ones16 = jnp.ones((NP, DEGW), jnp.float32)
    xpad = jnp.zeros((NP - NN, DM), jnp.float32)
    x2 = jnp.concatenate([x_i, xpad, x_j, xpad], axis=0)

    deg16 = _sc_deg(dst2d, ones16)
    disb = _dis(deg16)

    y = _first(x2, disb, W1)
    z = _sc_scatter(y, src2d, dst2d)
    y = _mid(z, disb, b1.reshape(1, DM), W2)
    z = _sc_scatter(y, src2d, dst2d)
    y = _mid(z, disb, b2.reshape(1, DM), W3)
    z = _sc_scatter(y, src2d, dst2d)
    h3, cs = _fin(z, disb, b3.reshape(1, DM))
    g2 = _att(h3, cs, att_W)[::8]

    t2 = jnp.transpose(ntn_T, (1, 0, 2)).reshape(DM, NSL * DM)
    mt = ntn_M.T
    nb_row = ntn_b.reshape(1, NSL)
    mws = (mW1, mb1.reshape(1, -1), mW2, mb2.reshape(1, -1),
           mW3, mb3.reshape(1, -1), mW4, mb4.reshape(1, -1),
           sW, sb.reshape(1, 1))
    out = _head(g2, t2, mt, nb_row, mws)
    return out[0]


# speedup vs baseline: 1.0414x; 1.0052x over previous
import functools

import jax
import jax.numpy as jnp
from jax import lax
from jax.experimental import pallas as pl
from jax.experimental.pallas import tpu as pltpu
from jax.experimental.pallas import tpu_sc as plsc

NN = 10000
NP = 10240
NE = 320000
EPAD = 327680
DM = 128
NTIL = 16
RPT = NP // NTIL
CHUNK = 128
INNER = 8
IROWS = EPAD // CHUNK
IPT = IROWS // NTIL
OUTER = IPT // INNER
NBUF = 2
LOOK = 1
QROWS = 40
QN = IPT // QROWS
DEGW = 16
NSL = 16
BLK = 1024
NB1 = NP // BLK
NBLK = 2 * NP // BLK

_mesh = functools.partial(
    plsc.VectorSubcoreMesh, core_axis_name="c", subcore_axis_name="s")


def _sc_deg_body(dst2d, ones_hbm, deg_out, idx_v, ones_v, deg_sh):
    c = lax.axis_index("c")
    s = lax.axis_index("s")
    r0 = s * RPT
    pltpu.sync_copy(ones_hbm.at[pl.ds(0, CHUNK)], ones_v)
    pltpu.sync_copy(ones_hbm.at[pl.ds(r0, RPT)], deg_sh.at[pl.ds(r0, RPT)])
    plsc.subcore_barrier()
    row0 = c * IROWS + s * IPT

    def outer(g, carry):
        pltpu.sync_copy(dst2d.at[pl.ds(row0 + g * INNER, INNER)], idx_v)
        for j in range(INNER):
            pltpu.sync_copy(ones_v, deg_sh.at[idx_v.at[j]], add=True)
        return carry

    lax.fori_loop(0, OUTER, outer, 0)
    plsc.subcore_barrier()
    pltpu.sync_copy(deg_sh.at[pl.ds(r0, RPT)],
                    deg_out.at[pl.ds(c * NP + r0, RPT)])


def _sc_deg(dst2d, ones_hbm):
    return pl.kernel(
        _sc_deg_body,
        out_type=jax.ShapeDtypeStruct((2 * NP, DEGW), jnp.float32),
        mesh=_mesh(),
        scratch_types=[
            pltpu.VMEM((INNER, CHUNK), jnp.int32),
            pltpu.VMEM((CHUNK, DEGW), jnp.float32),
            pltpu.VMEM_SHARED((NP, DEGW), jnp.float32),
        ],
    )(dst2d, ones_hbm)


def _sc_scat_body(y_hbm, src2d, dst2d, z_out, idxs, idxd, rows, z_sh,
                  gsem, ssem, isem):
    c = lax.axis_index("c")
    s = lax.axis_index("s")
    r0 = s * RPT
    base = c * NP
    pltpu.sync_copy(y_hbm.at[pl.ds(base + r0, RPT)], z_sh.at[pl.ds(r0, RPT)])
    row0 = c * IROWS + s * IPT
    plsc.subcore_barrier()

    def _gather(chunk, b):
        return pltpu.async_copy(y_hbm.at[idxs.at[chunk]], rows.at[b],
                                gsem.at[b])

    def _scat(chunk, b):
        return pltpu.async_copy(rows.at[b], z_sh.at[idxd.at[chunk]],
                                ssem.at[b], add=True)

    def window(q, carry):
        cs = pltpu.async_copy(src2d.at[pl.ds(row0 + q * QROWS, QROWS)],
                              idxs, isem)
        cd = pltpu.async_copy(dst2d.at[pl.ds(row0 + q * QROWS, QROWS)],
                              idxd, isem)

        @pl.when(q > 0)
        def _():
            for b in range(NBUF):
                pltpu.make_async_copy(rows.at[b], z_sh.at[idxd.at[0]],
                                      ssem.at[b]).wait()

        cs.wait()
        cd.wait()
        for b in range(LOOK):
            _gather(b, b)

        def outer(w, carry2):
            for b in range(NBUF):
                ch = w * NBUF + b
                pltpu.make_async_copy(y_hbm.at[idxs.at[ch]], rows.at[b],
                                      gsem.at[b]).wait()
                _scat(ch, b)
                nb = (b + LOOK) % NBUF
                nc = ch + LOOK

                @pl.when(jnp.logical_and(nc >= NBUF, nc < QROWS))
                def _():
                    pltpu.make_async_copy(rows.at[nb],
                                          z_sh.at[idxd.at[ch]],
                                          ssem.at[nb]).wait()
                    _gather(nc, nb)

                @pl.when(jnp.logical_and(nc >= LOOK, nc < NBUF))
                def _():
                    _gather(nc, nb)

            return carry2

        lax.fori_loop(0, QROWS // NBUF, outer, 0)
        return carry

    lax.fori_loop(0, QN, window, 0)
    for b in range(NBUF):
        pltpu.make_async_copy(rows.at[b], z_sh.at[idxd.at[0]],
                              ssem.at[b]).wait()
    plsc.subcore_barrier()
    pltpu.sync_copy(z_sh.at[pl.ds(r0, RPT)], z_out.at[pl.ds(base + r0, RPT)])


def _sc_scatter(y, src2d, dst2d):
    return pl.kernel(
        _sc_scat_body,
        out_type=jax.ShapeDtypeStruct((2 * NP, DM), jnp.float32),
        mesh=_mesh(),
        scratch_types=[
            pltpu.VMEM((QROWS, CHUNK), jnp.int32),
            pltpu.VMEM((QROWS, CHUNK), jnp.int32),
            pltpu.VMEM((NBUF, CHUNK, DM), jnp.float32),
            pltpu.VMEM_SHARED((NP, DM), jnp.float32),
            pltpu.SemaphoreType.DMA((NBUF,)),
            pltpu.SemaphoreType.DMA((NBUF,)),
            pltpu.SemaphoreType.DMA,
        ],
    )(y, src2d, dst2d)


def _dis_body(deg_ref, out_ref):
    d = deg_ref[:, 0:1]
    out_ref[...] = jnp.broadcast_to(lax.rsqrt(d), (BLK, DM))


def _dis(deg16):
    return pl.pallas_call(
        _dis_body,
        grid=(NBLK,),
        in_specs=[pl.BlockSpec((BLK, DEGW), lambda i: (i, 0))],
        out_specs=pl.BlockSpec((BLK, DM), lambda i: (i, 0)),
        out_shape=jax.ShapeDtypeStruct((2 * NP, DM), jnp.float32),
    )(deg16)


def _first_body(x_ref, dis_ref, w_ref, y_ref):
    y_ref[...] = dis_ref[...] * jnp.dot(
        x_ref[...], w_ref[...], preferred_element_type=jnp.float32)


def _first(x2, disb, w):
    return pl.pallas_call(
        _first_body,
        grid=(NBLK,),
        in_specs=[
            pl.BlockSpec((BLK, DM), lambda i: (i, 0)),
            pl.BlockSpec((BLK, DM), lambda i: (i, 0)),
            pl.BlockSpec((DM, DM), lambda i: (0, 0)),
        ],
        out_specs=pl.BlockSpec((BLK, DM), lambda i: (i, 0)),
        out_shape=jax.ShapeDtypeStruct((2 * NP, DM), jnp.float32),
    )(x2, disb, w)


def _mid_body(z_ref, dis_ref, b_ref, w_ref, y_ref):
    h = jax.nn.relu(dis_ref[...] * z_ref[...] + b_ref[...])
    y_ref[...] = dis_ref[...] * jnp.dot(
        h, w_ref[...], preferred_element_type=jnp.float32)


def _mid(z, disb, b_row, w):
    return pl.pallas_call(
        _mid_body,
        grid=(NBLK,),
        in_specs=[
            pl.BlockSpec((BLK, DM), lambda i: (i, 0)),
            pl.BlockSpec((BLK, DM), lambda i: (i, 0)),
            pl.BlockSpec((1, DM), lambda i: (0, 0)),
            pl.BlockSpec((DM, DM), lambda i: (0, 0)),
        ],
        out_specs=pl.BlockSpec((BLK, DM), lambda i: (i, 0)),
        out_shape=jax.ShapeDtypeStruct((2 * NP, DM), jnp.float32),
    )(z, disb, b_row, w)


def _valid_mask():
    row = pl.program_id(1) * BLK + lax.broadcasted_iota(
        jnp.int32, (BLK, 1), 0)
    return row < NN


def _fin_body(z_ref, dis_ref, b_ref, h_ref, cs_ref):
    h = dis_ref[...] * z_ref[...] + b_ref[...]
    h_ref[...] = h

    @pl.when(pl.program_id(1) == 0)
    def _():
        cs_ref[...] = jnp.zeros_like(cs_ref)

    hm = jnp.where(_valid_mask(), h, 0.0)
    cs_ref[...] += jnp.broadcast_to(jnp.sum(hm, axis=0, keepdims=True),
                                    (8, DM))


def _fin(z, disb, b_row):
    return pl.pallas_call(
        _fin_body,
        grid=(2, NB1),
        in_specs=[
            pl.BlockSpec((BLK, DM), lambda g, i: (g * NB1 + i, 0)),
            pl.BlockSpec((BLK, DM), lambda g, i: (g * NB1 + i, 0)),
            pl.BlockSpec((1, DM), lambda g, i: (0, 0)),
        ],
        out_specs=[
            pl.BlockSpec((BLK, DM), lambda g, i: (g * NB1 + i, 0)),
            pl.BlockSpec((8, DM), lambda g, i: (g, 0)),
        ],
        out_shape=[
            jax.ShapeDtypeStruct((2 * NP, DM), jnp.float32),
            jax.ShapeDtypeStruct((16, DM), jnp.float32),
        ],
    )(z, disb, b_row)


def _att_body(h_ref, cs_ref, wa_ref, g_ref):
    gc = jnp.tanh(jnp.dot(cs_ref[0:1, :] * (1.0 / NN), wa_ref[...],
                          preferred_element_type=jnp.float32))
    logits = jnp.sum(h_ref[...] * gc, axis=1, keepdims=True)
    aw = jnp.where(_valid_mask(), jax.nn.sigmoid(logits), 0.0)

    @pl.when(pl.program_id(1) == 0)
    def _():
        g_ref[...] = jnp.zeros_like(g_ref)

    g_ref[...] += jnp.broadcast_to(
        jnp.sum(h_ref[...] * aw, axis=0, keepdims=True), (8, DM))


def _att(h3, cs, att_w):
    return pl.pallas_call(
        _att_body,
        grid=(2, NB1),
        in_specs=[
            pl.BlockSpec((BLK, DM), lambda g, i: (g * NB1 + i, 0)),
            pl.BlockSpec((8, DM), lambda g, i: (g, 0)),
            pl.BlockSpec((DM, DM), lambda g, i: (0, 0)),
        ],
        out_specs=pl.BlockSpec((8, DM), lambda g, i: (g, 0)),
        out_shape=jax.ShapeDtypeStruct((16, DM), jnp.float32),
    )(h3, cs, att_w)


def _head_body(g_ref, t2_ref, mt_ref, nb_ref, w1_ref, b1_ref, w2_ref, b2_ref,
               w3_ref, b3_ref, w4_ref, b4_ref, sw_ref, sb_ref, out_ref):
    gi = g_ref[0:1, :]
    gj = g_ref[1:2, :]
    u = jnp.dot(gi, t2_ref[...], preferred_element_type=jnp.float32)
    lane = lax.broadcasted_iota(jnp.int32, (1, NSL), 1)
    s1 = jnp.zeros((1, NSL), jnp.float32)
    for k in range(NSL):
        sk = jnp.sum(u[:, k * DM:(k + 1) * DM] * gj, axis=1, keepdims=True)
        s1 = s1 + jnp.where(lane == k, sk, 0.0)
    s2 = (jnp.dot(gi, mt_ref[0:DM, :], preferred_element_type=jnp.float32)
          + jnp.dot(gj, mt_ref[DM:2 * DM, :],
                    preferred_element_type=jnp.float32))
    sc = jnp.tanh(s1 + s2 + nb_ref[...])
    h = jax.nn.relu(jnp.dot(sc, w1_ref[...],
                            preferred_element_type=jnp.float32) + b1_ref[...])
    h = jax.nn.relu(jnp.dot(h, w2_ref[...],
                            preferred_element_type=jnp.float32) + b2_ref[...])
    h = jax.nn.relu(jnp.dot(h, w3_ref[...],
                            preferred_element_type=jnp.float32) + b3_ref[...])
    h = jax.nn.relu(jnp.dot(h, w4_ref[...],
                            preferred_element_type=jnp.float32) + b4_ref[...])
    out_ref[...] = jax.nn.sigmoid(
        jnp.dot(h, sw_ref[...], preferred_element_type=jnp.float32)
        + sb_ref[...])


def _head(g2, t2, mt, nb_row, mws):
    return pl.pallas_call(
        _head_body,
        out_shape=jax.ShapeDtypeStruct((1, 1), jnp.float32),
    )(g2, t2, mt, nb_row, *mws)


def kernel(x_i, edge_index_i, x_j, edge_index_j, W1, b1, W2, b2, W3, b3,
           att_W, ntn_T, ntn_M, ntn_b, mW1, mb1, mW2, mb2, mW3, mb3,
           mW4, mb4, sW, sb):
    ei = edge_index_i.astype(jnp.int32)
    ej = edge_index_j.astype(jnp.int32)
    pad = jnp.full((EPAD - NE,), NN, jnp.int32)
    src2d = jnp.concatenate(
        [ei[0], pad, ej[0] + NP, pad + NP]).reshape(2 * IROWS, CHUNK)
    dst2d = jnp.concatenate(
        [ei[1], pad, ej[1], pad]).reshape(2 * IROWS, CHUNK)
    ones16 = jnp.ones((NP, DEGW), jnp.float32)
    xpad = jnp.zeros((NP - NN, DM), jnp.float32)
    x2 = jnp.concatenate([x_i, xpad, x_j, xpad], axis=0)

    deg16 = _sc_deg(dst2d, ones16)
    disb = _dis(deg16)

    y = _first(x2, disb, W1)
    z = _sc_scatter(y, src2d, dst2d)
    y = _mid(z, disb, b1.reshape(1, DM), W2)
    z = _sc_scatter(y, src2d, dst2d)
    y = _mid(z, disb, b2.reshape(1, DM), W3)
    z = _sc_scatter(y, src2d, dst2d)
    h3, cs = _fin(z, disb, b3.reshape(1, DM))
    g2 = _att(h3, cs, att_W)[::8]

    t2 = jnp.transpose(ntn_T, (1, 0, 2)).reshape(DM, NSL * DM)
    mt = ntn_M.T
    nb_row = ntn_b.reshape(1, NSL)
    mws = (mW1, mb1.reshape(1, -1), mW2, mb2.reshape(1, -1),
           mW3, mb3.reshape(1, -1), mW4, mb4.reshape(1, -1),
           sW, sb.reshape(1, 1))
    out = _head(g2, t2, mt, nb_row, mws)
    return out[0]
